# baseline (speedup 1.0000x reference)
import functools

import jax
import jax.numpy as jnp
from jax import lax
from jax.experimental import pallas as pl
from jax.experimental.pallas import tpu as pltpu
from jax.experimental.pallas import tpu_sc as plsc

D_MODEL = 64
MAXLEN = 200
LANES = 16
NUM_WORKERS = 32
CHUNK = 512
SUB = 64
GRP = 16
EPS = 1e-5


def _rsqrt_vec(v):
    i = lax.bitcast_convert_type(v, jnp.int32)
    i = jnp.int32(0x5F3759DF) - (i >> 1)
    y = lax.bitcast_convert_type(i, jnp.float32)
    for _ in range(3):
        y = y * (1.5 - 0.5 * v * y * y)
    return y


def _make_sc_kernel(n_rows):
    rows_per_w = n_rows // NUM_WORKERS
    n_chunks = rows_per_w // CHUNK
    mesh = plsc.VectorSubcoreMesh(core_axis_name="c", subcore_axis_name="s")

    @functools.partial(
        pl.kernel,
        mesh=mesh,
        compiler_params=pltpu.CompilerParams(use_tc_tiling_on_sc=False,
                                             needs_layout_passes=False),
        out_type=jax.ShapeDtypeStruct((n_rows, D_MODEL), jnp.float32),
        scratch_types=[
            pltpu.VMEM((CHUNK // SUB, SUB), jnp.int32),
            pltpu.VMEM((CHUNK, D_MODEL), jnp.float32),
            pltpu.VMEM((MAXLEN, D_MODEL), jnp.float32),
            pltpu.VMEM((2, D_MODEL), jnp.float32),
            pltpu.SemaphoreType.DMA,
        ],
    )
    def sc_embed(x_hbm, tok_hbm, pos_hbm, gam_hbm, bet_hbm, out_hbm,
                 idx_v, rows_v, pos_v, gb_v, sem):
        cid = lax.axis_index("c")
        sid = lax.axis_index("s")
        wid = sid * 2 + cid
        base = wid * rows_per_w

        pltpu.sync_copy(pos_hbm, pos_v)
        pltpu.sync_copy(gam_hbm, gb_v.at[0])
        pltpu.sync_copy(bet_hbm, gb_v.at[1])

        lane = lax.iota(jnp.int32, LANES)
        g_vec = [gb_v[0, pl.ds(LANES * j, LANES)] for j in range(D_MODEL // LANES)]
        b_vec = [gb_v[1, pl.ds(LANES * j, LANES)] for j in range(D_MODEL // LANES)]

        def chunk_fn(i, carry):
            row0 = base + i * CHUNK
            jbase = pl.multiple_of(row0 // SUB, 8)
            pltpu.sync_copy(x_hbm.at[pl.ds(jbase, CHUNK // SUB)], idx_v)
            copies = [
                pltpu.async_copy(tok_hbm.at[idx_v.at[j]],
                                 rows_v.at[pl.ds(j * SUB, SUB)], sem)
                for j in range(CHUNK // SUB)
            ]
            for c in copies:
                c.wait()

            @plsc.parallel_loop(0, CHUNK // GRP)
            def grp_fn(g):
                row_idx = lane + g * GRP
                pos_idx = lax.rem(row0 + g * GRP + lane, MAXLEN)
                zero = jnp.zeros((LANES,), jnp.float32)
                s_acc = [zero] * 4
                q_acc = [zero] * 4
                for d in range(D_MODEL):
                    col = jnp.full((LANES,), d, jnp.int32)
                    t = (plsc.load_gather(rows_v, [row_idx, col])
                         + plsc.load_gather(pos_v, [pos_idx, col]))
                    s_acc[d % 4] = s_acc[d % 4] + t
                    q_acc[d % 4] = q_acc[d % 4] + t * t
                s = (s_acc[0] + s_acc[1]) + (s_acc[2] + s_acc[3])
                q = (q_acc[0] + q_acc[1]) + (q_acc[2] + q_acc[3])
                mean = s * (1.0 / D_MODEL)
                var = jnp.maximum(q * (1.0 / D_MODEL) - mean * mean, 0.0)
                rstd = _rsqrt_vec(var + EPS)
                for d in range(D_MODEL):
                    col = jnp.full((LANES,), d, jnp.int32)
                    t = (plsc.load_gather(rows_v, [row_idx, col])
                         + plsc.load_gather(pos_v, [pos_idx, col]))
                    out = ((t - mean) * rstd * g_vec[d // LANES][d % LANES]
                           + b_vec[d // LANES][d % LANES])
                    plsc.store_scatter(rows_v, [row_idx, col], out)

            pltpu.sync_copy(rows_v, out_hbm.at[pl.ds(row0, CHUNK)])
            return carry

        lax.fori_loop(0, n_chunks, chunk_fn, 0)

    return sc_embed


def kernel(x, tok_table, pos_table, gamma, beta):
    bsz, seq = x.shape
    n_rows = bsz * seq
    assert n_rows % (NUM_WORKERS * CHUNK) == 0
    assert seq == MAXLEN and tok_table.shape[1] == D_MODEL
    x_flat = x.reshape(n_rows // SUB, SUB).astype(jnp.int32)
    sc = _make_sc_kernel(n_rows)
    out = sc(x_flat, tok_table, pos_table, gamma, beta)
    return out.reshape(bsz, seq, D_MODEL)
